# baseline (speedup 1.0000x reference)
import functools

import jax
import jax.numpy as jnp
from jax import lax
from jax.experimental import pallas as pl
from jax.experimental.pallas import tpu as pltpu


def _swap(arrs, j, keep):
    sp = arrs[0].shape[0]
    n = arrs[0].shape[-1]
    g = sp // (2 * j)
    outs = []
    for arr in arrs:
        h = arr.reshape(g, 2, j, n)
        a, b = h[:, 0], h[:, 1]
        na = jnp.where(keep, a, b)
        nb = jnp.where(keep, b, a)
        outs.append(
            jnp.concatenate([na[:, None], nb[:, None]], axis=1).reshape(sp, n)
        )
    return outs


def _stage_masks(arrs, j, k, sp, lt):
    g = sp // (2 * j)
    n = arrs[0].shape[-1]
    halves = [a.reshape(g, 2, j, n) for a in arrs]
    ah = [h[:, 0] for h in halves]
    bh = [h[:, 1] for h in halves]
    gi = lax.broadcasted_iota(jnp.int32, (g, 1, 1), 0)
    asc = ((gi * (2 * j)) & k) == 0
    keep = lt(ah, bh) == asc
    return _swap(arrs, j, keep), keep


def _bitonic(arrs, sp, lt):
    stages = []
    k = 2
    while k <= sp:
        j = k // 2
        while j >= 1:
            arrs, keep = _stage_masks(arrs, j, k, sp, lt)
            stages.append((j, keep))
            j //= 2
        k *= 2
    return arrs, stages


def _unsort(arr, stages):
    for j, keep in reversed(stages):
        (arr,) = _swap([arr], j, keep)
    return arr


def _body(idx_ref, x_ref, emp_ref, out_ref, ysel_ref, ytile_ref, *, s, sp, nt):
    b = pl.program_id(0)
    it = pl.program_id(1)
    n = emp_ref.shape[1]

    @pl.when(it == 0)
    def _():
        def gather_row(i, carry):
            t = idx_ref[b, i]
            ysel_ref[pl.ds(i, 1), :] = emp_ref[pl.ds(t, 1), :]
            return carry

        lax.fori_loop(0, s, gather_row, 0, unroll=8)
        yfull = ysel_ref[...]
        for c in range(n // nt):
            ytile_ref[c] = yfull[:, c * nt:(c + 1) * nt]

    inf = jnp.float32(jnp.inf)
    row = lax.broadcasted_iota(jnp.int32, (sp, nt), 0)

    xt = x_ref[0].T
    xpad = jnp.concatenate([xt, jnp.full((sp - s, nt), inf, jnp.float32)], axis=0)
    (sx,), _ = _bitonic([xpad], sp, lambda a, b: a[0] < b[0])

    y = jnp.where(row < s, ytile_ref[it], inf)
    _, stages = _bitonic(
        [y, row],
        sp,
        lambda a, b: (a[0] < b[0]) | ((a[0] == b[0]) & (a[1] < b[1])),
    )

    out_t = _unsort(sx, stages)

    out_ref[0] = out_t[:s].T


def kernel(out_sample_hat, indices, empirical_distribution):
    bsz, n, s = out_sample_hat.shape
    t = empirical_distribution.shape[0]
    sp = max(8, 1 << (s - 1).bit_length())
    nt = n
    for cand in (256, 200, 128):
        if n % cand == 0 and cand % 8 == 0:
            nt = cand
            break
    n_tiles = n // nt

    body = functools.partial(_body, s=s, sp=sp, nt=nt)
    grid_spec = pltpu.PrefetchScalarGridSpec(
        num_scalar_prefetch=1,
        grid=(bsz, n_tiles),
        in_specs=[
            pl.BlockSpec((1, nt, s), lambda b, i, idx: (b, i, 0)),
            pl.BlockSpec((t, n), lambda b, i, idx: (0, 0)),
        ],
        out_specs=pl.BlockSpec((1, nt, s), lambda b, i, idx: (b, i, 0)),
        scratch_shapes=[
            pltpu.VMEM((sp, n), jnp.float32),
            pltpu.VMEM((n_tiles, sp, nt), jnp.float32),
        ],
    )
    return pl.pallas_call(
        body,
        grid_spec=grid_spec,
        out_shape=jax.ShapeDtypeStruct((bsz, n, s), out_sample_hat.dtype),
        compiler_params=pltpu.CompilerParams(
            dimension_semantics=("arbitrary", "arbitrary"),
            vmem_limit_bytes=110 * 1024 * 1024,
        ),
    )(indices.astype(jnp.int32), out_sample_hat, empirical_distribution)


# speedup vs baseline: 4.9794x; 4.9794x over previous
import functools

import jax
import jax.numpy as jnp
from jax import lax
from jax.experimental import pallas as pl
from jax.experimental.pallas import tpu as pltpu


def _split(a, j, k, sp):
    m = a.shape[-1]
    q = max(sp // (2 * k), 1)
    d = 2 if k < sp else 1
    return a.reshape(q, d, k // (2 * j), 2, j, m)


def _join(asc, desc, sp):
    m = asc[0].shape[-1]
    blk = [jnp.concatenate([na[:, :, None], nb[:, :, None]], axis=2)
           for na, nb in (asc,) + ((desc,) if desc is not None else ())]
    full = blk[0][:, None] if desc is None else jnp.concatenate(
        [blk[0][:, None], blk[1][:, None]], axis=1)
    return full.reshape(sp, m)


def _mm_stage(a, j, k, sp):
    r = _split(a, j, k, sp)
    aa, ab = r[:, 0, :, 0], r[:, 0, :, 1]
    asc = (jnp.minimum(aa, ab), jnp.maximum(aa, ab))
    desc = None
    if k < sp:
        da, db = r[:, 1, :, 0], r[:, 1, :, 1]
        desc = (jnp.maximum(da, db), jnp.minimum(da, db))
    return _join(asc, desc, sp)


def _lex_lt(ya, ta, yb, tb):
    return (ya < yb) | ((ya == yb) & (ta < tb))


def _lex_stage(y, t, j, k, sp):
    ry = _split(y, j, k, sp)
    rt = _split(t, j, k, sp)
    ya, yb = ry[:, 0, :, 0], ry[:, 0, :, 1]
    ta, tb = rt[:, 0, :, 0], rt[:, 0, :, 1]
    ka = _lex_lt(ya, ta, yb, tb)
    asc_y = (jnp.where(ka, ya, yb), jnp.where(ka, yb, ya))
    asc_t = (jnp.where(ka, ta, tb), jnp.where(ka, tb, ta))
    desc_y = desc_t = kd = None
    if k < sp:
        ya, yb = ry[:, 1, :, 0], ry[:, 1, :, 1]
        ta, tb = rt[:, 1, :, 0], rt[:, 1, :, 1]
        kd = _lex_lt(yb, tb, ya, ta)
        desc_y = (jnp.where(kd, ya, yb), jnp.where(kd, yb, ya))
        desc_t = (jnp.where(kd, ta, tb), jnp.where(kd, tb, ta))
    return (_join(asc_y, desc_y, sp), _join(asc_t, desc_t, sp)), (ka, kd)


def _apply_stage(a, j, k, sp, masks):
    ka, kd = masks
    r = _split(a, j, k, sp)
    aa, ab = r[:, 0, :, 0], r[:, 0, :, 1]
    asc = (jnp.where(ka, aa, ab), jnp.where(ka, ab, aa))
    desc = None
    if kd is not None:
        da, db = r[:, 1, :, 0], r[:, 1, :, 1]
        desc = (jnp.where(kd, da, db), jnp.where(kd, db, da))
    return _join(asc, desc, sp)


def _stages(sp):
    out = []
    k = 2
    while k <= sp:
        j = k // 2
        while j >= 1:
            out.append((j, k))
            j //= 2
        k *= 2
    return out


def _body(idx_ref, x_ref, emp_ref, out_ref, ysel_ref, *, s, sp, n):
    b = pl.program_id(0)

    def gather_row(i, carry):
        t = idx_ref[b, i]
        ysel_ref[pl.ds(i, 1), :] = emp_ref[pl.ds(t, 1), :]
        return carry

    lax.fori_loop(0, s, gather_row, 0, unroll=8)

    inf = jnp.float32(jnp.inf)
    row = lax.broadcasted_iota(jnp.int32, (sp, n), 0)

    xt = x_ref[0].T
    sx = jnp.concatenate([xt, jnp.full((sp - s, n), inf, jnp.float32)], axis=0)
    for j, k in _stages(sp):
        sx = _mm_stage(sx, j, k, sp)

    y = jnp.where(row < s, ysel_ref[...], inf)
    t = row
    recorded = []
    for j, k in _stages(sp):
        (y, t), masks = _lex_stage(y, t, j, k, sp)
        recorded.append((j, k, masks))

    for j, k, masks in reversed(recorded):
        sx = _apply_stage(sx, j, k, sp, masks)

    out_ref[0] = sx[:s].T


def kernel(out_sample_hat, indices, empirical_distribution):
    bsz, n, s = out_sample_hat.shape
    t = empirical_distribution.shape[0]
    sp = max(8, 1 << (s - 1).bit_length())

    body = functools.partial(_body, s=s, sp=sp, n=n)
    grid_spec = pltpu.PrefetchScalarGridSpec(
        num_scalar_prefetch=1,
        grid=(bsz,),
        in_specs=[
            pl.BlockSpec((1, n, s), lambda b, idx: (b, 0, 0)),
            pl.BlockSpec((t, n), lambda b, idx: (0, 0)),
        ],
        out_specs=pl.BlockSpec((1, n, s), lambda b, idx: (b, 0, 0)),
        scratch_shapes=[pltpu.VMEM((sp, n), jnp.float32)],
    )
    return pl.pallas_call(
        body,
        grid_spec=grid_spec,
        out_shape=jax.ShapeDtypeStruct((bsz, n, s), out_sample_hat.dtype),
        compiler_params=pltpu.CompilerParams(
            dimension_semantics=("arbitrary",),
            vmem_limit_bytes=110 * 1024 * 1024,
        ),
    )(indices.astype(jnp.int32), out_sample_hat, empirical_distribution)
